# Initial kernel scaffold; baseline (speedup 1.0000x reference)
import jax
import jax.numpy as jnp
from jax.experimental import pallas as pl


def kernel(x, W, b):
    raise NotImplementedError("write your pallas kernel here")


# speedup vs baseline: 1.2415x; 1.2415x over previous
import jax
import jax.numpy as jnp
from jax.experimental import pallas as pl
from jax.experimental.pallas import tpu as pltpu

B = 128
D_IN = 128
NUM_ATTRS = 8
NUM_ELEMS = 32768
TILE = 4096
TILES = NUM_ELEMS // TILE


def _sampler_kernel(x_ref, w_ref, b_ref, logits_ref, idx_ref, max_s, arg_s):
    i = pl.program_id(0)
    k = pl.program_id(1)
    vals = jnp.dot(x_ref[...], w_ref[...], preferred_element_type=jnp.float32)
    vals = vals + b_ref[...]
    logits_ref[...] = vals

    m = jnp.max(vals, axis=1, keepdims=True)
    a = jnp.argmax(vals, axis=1).astype(jnp.int32)[:, None] + k * TILE

    @pl.when(k == 0)
    def _():
        max_s[...] = m
        arg_s[...] = a

    @pl.when(k != 0)
    def _():
        better = m > max_s[...]
        arg_s[...] = jnp.where(better, a, arg_s[...])
        max_s[...] = jnp.where(better, m, max_s[...])

    @pl.when(k == TILES - 1)
    def _():
        lane = jax.lax.broadcasted_iota(jnp.int32, (B, NUM_ATTRS), 1)
        idx_ref[...] = jnp.where(lane == i, arg_s[...], idx_ref[...])


def kernel(x, W, b):
    b2 = b.reshape(1, NUM_ATTRS * NUM_ELEMS)
    logits_flat, idx = pl.pallas_call(
        _sampler_kernel,
        grid=(NUM_ATTRS, TILES),
        in_specs=[
            pl.BlockSpec((B, D_IN), lambda i, k: (0, 0)),
            pl.BlockSpec((D_IN, TILE), lambda i, k: (0, i * TILES + k)),
            pl.BlockSpec((1, TILE), lambda i, k: (0, i * TILES + k)),
        ],
        out_specs=[
            pl.BlockSpec((B, TILE), lambda i, k: (0, i * TILES + k)),
            pl.BlockSpec((B, NUM_ATTRS), lambda i, k: (0, 0)),
        ],
        out_shape=[
            jax.ShapeDtypeStruct((B, NUM_ATTRS * NUM_ELEMS), jnp.float32),
            jax.ShapeDtypeStruct((B, NUM_ATTRS), jnp.int32),
        ],
        scratch_shapes=[
            pltpu.VMEM((B, 1), jnp.float32),
            pltpu.VMEM((B, 1), jnp.int32),
        ],
        compiler_params=pltpu.CompilerParams(
            dimension_semantics=("arbitrary", "arbitrary"),
        ),
    )(x, W, b2)
    return idx, logits_flat.reshape(B, NUM_ATTRS, NUM_ELEMS)


# speedup vs baseline: 1.3489x; 1.0864x over previous
import jax
import jax.numpy as jnp
from jax.experimental import pallas as pl
from jax.experimental.pallas import tpu as pltpu

B = 128
D_IN = 128
NUM_ATTRS = 8
NUM_ELEMS = 32768
TILE = 8192
TILES = NUM_ELEMS // TILE


def _sampler_kernel(x_ref, w_ref, b_ref, logits_ref, idx_ref, max_s, arg_s):
    i = pl.program_id(0)
    k = pl.program_id(1)
    vals = jnp.dot(x_ref[...], w_ref[...], preferred_element_type=jnp.float32)
    vals = vals + b_ref[...]
    logits_ref[...] = vals

    m = jnp.max(vals, axis=1, keepdims=True)
    a = jnp.argmax(vals, axis=1).astype(jnp.int32)[:, None] + k * TILE

    @pl.when(k == 0)
    def _():
        max_s[...] = m
        arg_s[...] = a

    @pl.when(k != 0)
    def _():
        better = m > max_s[...]
        arg_s[...] = jnp.where(better, a, arg_s[...])
        max_s[...] = jnp.where(better, m, max_s[...])

    @pl.when(k == TILES - 1)
    def _():
        lane = jax.lax.broadcasted_iota(jnp.int32, (B, NUM_ATTRS), 1)
        idx_ref[...] = jnp.where(lane == i, arg_s[...], idx_ref[...])


def kernel(x, W, b):
    b2 = b.reshape(1, NUM_ATTRS * NUM_ELEMS)
    logits_flat, idx = pl.pallas_call(
        _sampler_kernel,
        grid=(NUM_ATTRS, TILES),
        in_specs=[
            pl.BlockSpec((B, D_IN), lambda i, k: (0, 0)),
            pl.BlockSpec((D_IN, TILE), lambda i, k: (0, i * TILES + k)),
            pl.BlockSpec((1, TILE), lambda i, k: (0, i * TILES + k)),
        ],
        out_specs=[
            pl.BlockSpec((B, TILE), lambda i, k: (0, i * TILES + k)),
            pl.BlockSpec((B, NUM_ATTRS), lambda i, k: (0, 0)),
        ],
        out_shape=[
            jax.ShapeDtypeStruct((B, NUM_ATTRS * NUM_ELEMS), jnp.float32),
            jax.ShapeDtypeStruct((B, NUM_ATTRS), jnp.int32),
        ],
        scratch_shapes=[
            pltpu.VMEM((B, 1), jnp.float32),
            pltpu.VMEM((B, 1), jnp.int32),
        ],
        compiler_params=pltpu.CompilerParams(
            dimension_semantics=("arbitrary", "arbitrary"),
        ),
    )(x, W, b2)
    return idx, logits_flat.reshape(B, NUM_ATTRS, NUM_ELEMS)


# speedup vs baseline: 1.3607x; 1.0088x over previous
import jax
import jax.numpy as jnp
from jax.experimental import pallas as pl
from jax.experimental.pallas import tpu as pltpu

B = 128
D_IN = 128
NUM_ATTRS = 8
NUM_ELEMS = 32768
TILE = 16384
TILES = NUM_ELEMS // TILE


def _sampler_kernel(x_ref, w_ref, b_ref, logits_ref, idx_ref, max_s, arg_s):
    i = pl.program_id(0)
    k = pl.program_id(1)
    vals = jnp.dot(x_ref[...], w_ref[...], preferred_element_type=jnp.float32)
    vals = vals + b_ref[...]
    logits_ref[...] = vals

    m = jnp.max(vals, axis=1, keepdims=True)
    a = jnp.argmax(vals, axis=1).astype(jnp.int32)[:, None] + k * TILE

    @pl.when(k == 0)
    def _():
        max_s[...] = m
        arg_s[...] = a

    @pl.when(k != 0)
    def _():
        better = m > max_s[...]
        arg_s[...] = jnp.where(better, a, arg_s[...])
        max_s[...] = jnp.where(better, m, max_s[...])

    @pl.when(k == TILES - 1)
    def _():
        lane = jax.lax.broadcasted_iota(jnp.int32, (B, NUM_ATTRS), 1)
        idx_ref[...] = jnp.where(lane == i, arg_s[...], idx_ref[...])


def kernel(x, W, b):
    b2 = b.reshape(1, NUM_ATTRS * NUM_ELEMS)
    logits_flat, idx = pl.pallas_call(
        _sampler_kernel,
        grid=(NUM_ATTRS, TILES),
        in_specs=[
            pl.BlockSpec((B, D_IN), lambda i, k: (0, 0)),
            pl.BlockSpec((D_IN, TILE), lambda i, k: (0, i * TILES + k)),
            pl.BlockSpec((1, TILE), lambda i, k: (0, i * TILES + k)),
        ],
        out_specs=[
            pl.BlockSpec((B, TILE), lambda i, k: (0, i * TILES + k)),
            pl.BlockSpec((B, NUM_ATTRS), lambda i, k: (0, 0)),
        ],
        out_shape=[
            jax.ShapeDtypeStruct((B, NUM_ATTRS * NUM_ELEMS), jnp.float32),
            jax.ShapeDtypeStruct((B, NUM_ATTRS), jnp.int32),
        ],
        scratch_shapes=[
            pltpu.VMEM((B, 1), jnp.float32),
            pltpu.VMEM((B, 1), jnp.int32),
        ],
        compiler_params=pltpu.CompilerParams(
            dimension_semantics=("arbitrary", "arbitrary"),
        ),
    )(x, W, b2)
    return idx, logits_flat.reshape(B, NUM_ATTRS, NUM_ELEMS)
